# baseline (speedup 1.0000x reference)
import functools

import jax
import jax.numpy as jnp
from jax import lax
from jax.experimental import pallas as pl
from jax.experimental.pallas import tpu as pltpu
from jax.experimental.pallas import tpu_sc as plsc

_B, _T, _N_IN, _N_OUT, _K = 8, 1024, 96, 32, 8192
_BT = _B * _T

_TM = 1024
_KC = 2048


def _argmin_body(x_ref, w_ref, b_ref, cb_ref, idx_ref):
    xt = x_ref[...]
    wt = w_ref[...]
    out = lax.dot_general(xt, wt, (((1,), (1,)), ((), ())),
                          preferred_element_type=jnp.float32)
    out = out + b_ref[0, :][None, :]
    out2 = jnp.sum(out * out, axis=1, keepdims=True)
    cb = cb_ref[...]
    c2 = jnp.sum(cb * cb, axis=1)[None, :]
    s = lax.dot_general(-2.0 * out, cb, (((1,), (1,)), ((), ())),
                        preferred_element_type=jnp.float32)
    d = (out2 + s) + c2
    idx_ref[...] = jnp.argmin(d, axis=1).astype(jnp.int32).reshape(1, 1, _TM)


def _nearest_idx(x2d, w, b2d, cb):
    nt = x2d.shape[0]
    return pl.pallas_call(
        _argmin_body,
        grid=(nt // _TM,),
        in_specs=[
            pl.BlockSpec((_TM, _N_IN), lambda i: (i, 0)),
            pl.BlockSpec((_N_OUT, _N_IN), lambda i: (0, 0)),
            pl.BlockSpec((1, _N_OUT), lambda i: (0, 0)),
            pl.BlockSpec((_K, _N_OUT), lambda i: (0, 0)),
        ],
        out_specs=pl.BlockSpec((1, 1, _TM), lambda i: (i, 0, 0)),
        out_shape=jax.ShapeDtypeStruct((nt // _TM, 1, _TM), jnp.int32),
    )(x2d, w, b2d, cb)


_NC, _NS = 2, 16
_NW = _NC * _NS
_CH = 128


def _sc_gather(cb, idx2d):
    nt = idx2d.shape[0] * idx2d.shape[1]
    bpw = nt // _NW
    nch = bpw // _CH

    def body(cb_hbm, idx_hbm, out_hbm, idx_v, rows_v, sem):
        wid = lax.axis_index("s") * _NC + lax.axis_index("c")
        pltpu.sync_copy(idx_hbm.at[pl.ds(wid * nch, nch)], idx_v)
        copies = [
            pltpu.async_copy(cb_hbm.at[idx_v.at[j]],
                             rows_v.at[pl.ds(j * _CH, _CH)], sem)
            for j in range(nch)
        ]
        for cp in copies:
            cp.wait()
        pltpu.sync_copy(rows_v, out_hbm.at[pl.ds(wid * bpw, bpw)])

    mesh = plsc.VectorSubcoreMesh(core_axis_name="c", subcore_axis_name="s")
    run = pl.kernel(
        body,
        out_type=jax.ShapeDtypeStruct((nt, _N_OUT), jnp.float32),
        mesh=mesh,
        scratch_types=[
            pltpu.VMEM((nch, _CH), jnp.int32),
            pltpu.VMEM((bpw, _N_OUT), jnp.float32),
            pltpu.SemaphoreType.DMA,
        ],
        compiler_params=pltpu.CompilerParams(use_tc_tiling_on_sc=False),
    )
    return run(cb, idx2d)


def kernel(x, W, b, codebook):
    x2d = x.reshape(_BT, _N_IN)
    b2d = b.reshape(1, _N_OUT)
    half = _BT // 2
    idx_a = _nearest_idx(x2d[:half], W, b2d, codebook)
    idx_b = _nearest_idx(x2d[half:], W, b2d, codebook)
    quant_a = _sc_gather(codebook, idx_a.reshape(half // _CH, _CH))
    quant_b = _sc_gather(codebook, idx_b.reshape(half // _CH, _CH))
    quant = jnp.concatenate([quant_a, quant_b], axis=0)
    return quant.reshape(_B, _T, _N_OUT)


# speedup vs baseline: 1.0472x; 1.0472x over previous
import functools

import jax
import jax.numpy as jnp
from jax import lax
from jax.experimental import pallas as pl
from jax.experimental.pallas import tpu as pltpu
from jax.experimental.pallas import tpu_sc as plsc

_B, _T, _N_IN, _N_OUT, _K = 8, 1024, 96, 32, 8192
_BT = _B * _T

_TM = 2048
_KC = 2048


def _argmin_body(x_ref, w_ref, b_ref, cb_ref, idx_ref):
    xt = x_ref[...]
    wt = w_ref[...]
    out = lax.dot_general(xt, wt, (((1,), (1,)), ((), ())),
                          preferred_element_type=jnp.float32)
    out = out + b_ref[0, :][None, :]
    out2 = jnp.sum(out * out, axis=1, keepdims=True)
    cb = cb_ref[...]
    c2 = jnp.sum(cb * cb, axis=1)[None, :]
    s = lax.dot_general(-2.0 * out, cb, (((1,), (1,)), ((), ())),
                        preferred_element_type=jnp.float32)
    d = (out2 + s) + c2
    idx_ref[...] = jnp.argmin(d, axis=1).astype(jnp.int32).reshape(1, 1, _TM)


def _nearest_idx(x2d, w, b2d, cb):
    nt = x2d.shape[0]
    return pl.pallas_call(
        _argmin_body,
        grid=(nt // _TM,),
        in_specs=[
            pl.BlockSpec((_TM, _N_IN), lambda i: (i, 0)),
            pl.BlockSpec((_N_OUT, _N_IN), lambda i: (0, 0)),
            pl.BlockSpec((1, _N_OUT), lambda i: (0, 0)),
            pl.BlockSpec((_K, _N_OUT), lambda i: (0, 0)),
        ],
        out_specs=pl.BlockSpec((1, 1, _TM), lambda i: (i, 0, 0)),
        out_shape=jax.ShapeDtypeStruct((nt // _TM, 1, _TM), jnp.int32),
        compiler_params=pltpu.CompilerParams(vmem_limit_bytes=120 * 1024 * 1024),
    )(x2d, w, b2d, cb)


_NC, _NS = 2, 16
_NW = _NC * _NS
_CH = 128


def _sc_gather(cb, idx2d):
    nt = idx2d.shape[0] * idx2d.shape[1]
    bpw = nt // _NW
    nch = bpw // _CH

    def body(cb_hbm, idx_hbm, out_hbm, idx_v, rows_v, sem):
        wid = lax.axis_index("s") * _NC + lax.axis_index("c")
        pltpu.sync_copy(idx_hbm.at[pl.ds(wid * nch, nch)], idx_v)
        copies = [
            pltpu.async_copy(cb_hbm.at[idx_v.at[j]],
                             rows_v.at[pl.ds(j * _CH, _CH)], sem)
            for j in range(nch)
        ]
        for cp in copies:
            cp.wait()
        pltpu.sync_copy(rows_v, out_hbm.at[pl.ds(wid * bpw, bpw)])

    mesh = plsc.VectorSubcoreMesh(core_axis_name="c", subcore_axis_name="s")
    run = pl.kernel(
        body,
        out_type=jax.ShapeDtypeStruct((nt, _N_OUT), jnp.float32),
        mesh=mesh,
        scratch_types=[
            pltpu.VMEM((nch, _CH), jnp.int32),
            pltpu.VMEM((bpw, _N_OUT), jnp.float32),
            pltpu.SemaphoreType.DMA,
        ],
        compiler_params=pltpu.CompilerParams(use_tc_tiling_on_sc=False),
    )
    return run(cb, idx2d)


def kernel(x, W, b, codebook):
    x2d = x.reshape(_BT, _N_IN)
    b2d = b.reshape(1, _N_OUT)
    idx = _nearest_idx(x2d, W, b2d, codebook)
    quant = _sc_gather(codebook, idx.reshape(_BT // _CH, _CH))
    return quant.reshape(_B, _T, _N_OUT)


# speedup vs baseline: 1.0738x; 1.0254x over previous
import jax
import jax.numpy as jnp
from jax import lax
from jax.experimental import pallas as pl
from jax.experimental.pallas import tpu as pltpu
from jax.experimental.pallas import tpu_sc as plsc

_B, _T, _N_IN, _N_OUT, _K = 8, 1024, 96, 32, 8192
_BT = _B * _T

_TM = 1024


def _argmin_body(x_ref, w_ref, b_ref, cb_ref, idx_ref):
    xt = x_ref[...]
    wt = w_ref[...]
    out = lax.dot_general(xt, wt, (((1,), (1,)), ((), ())),
                          preferred_element_type=jnp.float32)
    out = out + b_ref[0, :][None, :]
    out2 = jnp.sum(out * out, axis=1, keepdims=True)
    cb = cb_ref[...]
    c2 = jnp.sum(cb * cb, axis=1)[None, :]
    s = lax.dot_general(-2.0 * out, cb, (((1,), (1,)), ((), ())),
                        preferred_element_type=jnp.float32)
    d = (out2 + s) + c2
    idx_ref[...] = jnp.argmin(d, axis=1).astype(jnp.int32).reshape(1, 1, _TM)


def _nearest_idx(x2d, w, b2d, cb):
    nt = x2d.shape[0]
    return pl.pallas_call(
        _argmin_body,
        grid=(nt // _TM,),
        in_specs=[
            pl.BlockSpec((_TM, _N_IN), lambda i: (i, 0)),
            pl.BlockSpec((_N_OUT, _N_IN), lambda i: (0, 0)),
            pl.BlockSpec((1, _N_OUT), lambda i: (0, 0)),
            pl.BlockSpec((_K, _N_OUT), lambda i: (0, 0)),
        ],
        out_specs=pl.BlockSpec((1, 1, _TM), lambda i: (i, 0, 0)),
        out_shape=jax.ShapeDtypeStruct((nt // _TM, 1, _TM), jnp.int32),
    )(x2d, w, b2d, cb)


_NC, _NS = 2, 16
_NW = _NC * _NS
_CH = 128


def _sc_gather(cb, idx2d):
    nt = idx2d.shape[0] * idx2d.shape[1]
    bpw = nt // _NW
    nch = bpw // _CH

    def body(cb_hbm, idx_hbm, out_hbm, idx_v, rows_v, sem):
        wid = lax.axis_index("s") * _NC + lax.axis_index("c")
        pltpu.sync_copy(idx_hbm.at[pl.ds(wid * nch, nch)], idx_v)
        copies = [
            pltpu.async_copy(cb_hbm.at[idx_v.at[j]],
                             rows_v.at[pl.ds(j * _CH, _CH)], sem)
            for j in range(nch)
        ]
        for cp in copies:
            cp.wait()
        pltpu.sync_copy(rows_v, out_hbm.at[pl.ds(wid * bpw, bpw)])

    mesh = plsc.VectorSubcoreMesh(core_axis_name="c", subcore_axis_name="s")
    run = pl.kernel(
        body,
        out_type=jax.ShapeDtypeStruct((nt, _N_OUT), jnp.float32),
        mesh=mesh,
        scratch_types=[
            pltpu.VMEM((nch, _CH), jnp.int32),
            pltpu.VMEM((bpw, _N_OUT), jnp.float32),
            pltpu.SemaphoreType.DMA,
        ],
        compiler_params=pltpu.CompilerParams(use_tc_tiling_on_sc=False),
    )
    return run(cb, idx2d)


def kernel(x, W, b, codebook):
    x2d = x.reshape(_BT, _N_IN)
    b2d = b.reshape(1, _N_OUT)
    idx = _nearest_idx(x2d, W, b2d, codebook)
    quant = _sc_gather(codebook, idx.reshape(_BT // _CH, _CH))
    return quant.reshape(_B, _T, _N_OUT)
